# baseline (speedup 1.0000x reference)
import functools

import jax
import jax.numpy as jnp
from jax import lax
from jax.experimental import pallas as pl
from jax.experimental.pallas import tpu as pltpu

_THRESH = (128.0 / 255.0 - 0.5) / 0.5
_N = 256 * 126 * 126
_SCALE = 1.0 / (512.0 * 256.0 * 3.0) / (float(_N) * float(_N))


def _roll(x, shift, axis):
    return jnp.roll(x, shift, axis=axis)


def _accum(acc, av_ref, a, v, sign):
    av_ref[0:256, :] = v.reshape(256, 128).astype(jnp.int32)
    av_ref[256:512, :] = a.reshape(256, 128).astype(jnp.int32)
    lane_iota = lax.broadcasted_iota(jnp.int32, (1, 1, 128), 2)
    grp_iota = lax.broadcasted_iota(jnp.int32, (1, 1, 4), 2)
    rows = 64

    def chunk(i, c):
        vc = av_ref[pl.ds(i * rows, rows), :]
        ac = av_ref[pl.ds(256 + i * rows, rows), :]
        B = (vc[:, :, None] == lane_iota).astype(jnp.bfloat16)
        B = B.reshape(rows * 128, 128)
        A = (ac[:, :, None] == grp_iota).astype(jnp.bfloat16)
        A = A.reshape(rows * 128, 4)
        if sign < 0:
            A = -A
        d = lax.dot_general(A, B, (((0,), (0,)), ((), ())),
                            preferred_element_type=jnp.float32)
        return c + d

    return lax.fori_loop(0, 256 // rows, chunk, acc)


def _pattern_kernel(inp_ref, tgt_ref, out_ref, av_ref):
    ix = lax.broadcasted_iota(jnp.int32, (2, 128, 128), 2)
    iy = lax.broadcasted_iota(jnp.int32, (2, 128, 128), 1)
    mx = ix < 126
    my = iy < 126
    sentinel = jnp.float32(300.0)

    def slab(ref, z):
        zz = jnp.minimum(z, 127)
        s = ref[:, pl.ds(zz, 1), :, :].reshape(2, 128, 128)
        return (s >= _THRESH).astype(jnp.float32)

    def step(z, accs):
        axy, ayz, azx = accs
        zok = z <= 125
        m_xy = mx & my
        m_yz = mx & zok
        m_zx = my & zok
        for ref, sign in ((inp_ref, 1.0), (tgt_ref, -1.0)):
            Z0 = slab(ref, z)
            Z1 = slab(ref, z + 1)
            Z2 = slab(ref, z + 2)
            x1_0 = _roll(Z0, -1, 2)
            x2_0 = _roll(Z0, -2, 2)
            r0 = 4.0 * Z0 + 2.0 * x1_0 + x2_0
            a_xy = 2.0 * Z0 + x1_0
            v_xy = 64.0 * x2_0 + 8.0 * _roll(r0, -1, 1) + _roll(r0, -2, 1)
            v_xy = jnp.where(m_xy, v_xy, sentinel)
            axy = _accum(axy, av_ref, a_xy, v_xy, sign)
            x1_1 = _roll(Z1, -1, 2)
            x2_1 = _roll(Z1, -2, 2)
            x1_2 = _roll(Z2, -1, 2)
            x2_2 = _roll(Z2, -2, 2)
            r1 = 4.0 * Z1 + 2.0 * x1_1 + x2_1
            r2 = 4.0 * Z2 + 2.0 * x1_2 + x2_2
            v_yz = 64.0 * x2_0 + 8.0 * r1 + r2
            v_yz = jnp.where(m_yz, v_yz, sentinel)
            ayz = _accum(ayz, av_ref, a_xy, v_yz, sign)
            y1_0 = _roll(Z0, -1, 1)
            y2_0 = _roll(Z0, -2, 1)
            y1_1 = _roll(Z1, -1, 1)
            y2_1 = _roll(Z1, -2, 1)
            y1_2 = _roll(Z2, -1, 1)
            y2_2 = _roll(Z2, -2, 1)
            t1 = 4.0 * Z1 + 2.0 * y1_1 + y2_1
            t2 = 4.0 * Z2 + 2.0 * y1_2 + y2_2
            a_zx = 2.0 * Z0 + y1_0
            v_zx = 64.0 * y2_0 + 8.0 * t1 + t2
            v_zx = jnp.where(m_zx, v_zx, sentinel)
            azx = _accum(azx, av_ref, a_zx, v_zx, sign)
        return (axy, ayz, azx)

    zero = jnp.zeros((4, 128), jnp.float32)
    axy, ayz, azx = lax.fori_loop(0, 128, step, (zero, zero, zero))
    total = (jnp.sum(axy * axy) + jnp.sum(ayz * ayz) + jnp.sum(azx * azx))
    out_ref[:, :] = (total * _SCALE).reshape(1, 1)


@jax.jit
def kernel(input, target):
    out = pl.pallas_call(
        _pattern_kernel,
        out_shape=jax.ShapeDtypeStruct((1, 1), jnp.float32),
        scratch_shapes=[pltpu.VMEM((512, 128), jnp.int32)],
    )(input, target)
    return out[0, 0]


# speedup vs baseline: 19.5843x; 19.5843x over previous
import jax
import jax.numpy as jnp
from jax import lax
from jax.experimental import pallas as pl
from jax.experimental.pallas import tpu as pltpu

_THRESH = (128.0 / 255.0 - 0.5) / 0.5
_N = 256 * 126 * 126
_SCALE = 1.0 / (512.0 * 256.0 * 3.0) / (float(_N) * float(_N))
_K = 2 * 128 * 128
_BF = jnp.bfloat16


def _pattern_kernel(inp_ref, tgt_ref, out_ref):
    iv = lax.broadcasted_iota(jnp.int32, (32, _K), 0).astype(_BF)
    ia = lax.broadcasted_iota(jnp.int32, (16, _K), 0).astype(_BF)
    one = _BF(1)
    zero = _BF(0)

    ix = lax.broadcasted_iota(jnp.int32, (2, 128, 128), 2)
    iy = lax.broadcasted_iota(jnp.int32, (2, 128, 128), 1)
    mxy_f = jnp.where((ix < 126) & (iy < 126), 1.0, 0.0).astype(_BF)
    mx_f = jnp.where(ix < 126, 1.0, 0.0).astype(_BF)
    my_f = jnp.where(iy < 126, 1.0, 0.0).astype(_BF)
    sxy_f = (one - mxy_f) * _BF(33)
    sx_f = (one - mx_f) * _BF(33)
    sy_f = (one - my_f) * _BF(33)

    def slab(ref, z):
        zz = jnp.minimum(z, 127)
        s = ref[:, pl.ds(zz, 1), :, :].reshape(2, 128, 128)
        return jnp.where(s >= _THRESH, 1.0, 0.0).astype(_BF)

    def onehot_accum(acc, a, v, sign):
        fv = jnp.broadcast_to(v.reshape(1, _K), (32, _K))
        fa = jnp.broadcast_to(a.reshape(1, _K), (16, _K))
        B = jnp.where(fv == iv, one, zero)
        A = jnp.where(fa == ia, _BF(sign), zero)
        d = lax.dot_general(B, A, (((1,), (1,)), ((), ())),
                            preferred_element_type=jnp.float32)
        return acc + d

    def step(z, accs):
        axy, ayz, azx = accs
        mzf = jnp.where(z <= 125, 1.0, 0.0)
        mz = mzf.astype(_BF)
        sz = ((1.0 - mzf) * 33.0).astype(_BF)
        for ref, sign in ((inp_ref, 1.0), (tgt_ref, -1.0)):
            Z0 = slab(ref, z)
            Z1 = slab(ref, z + 1)
            Z2 = slab(ref, z + 2)
            rx1_0 = jnp.roll(Z0, -1, 2)
            rx2_0 = jnp.roll(Z0, -2, 2)
            ry1_0 = jnp.roll(Z0, -1, 1)
            ry2_0 = jnp.roll(Z0, -2, 1)
            a_xy = _BF(8) * Z0 + _BF(4) * rx1_0 + _BF(2) * rx2_0 + ry1_0
            v_xy = (_BF(16) * jnp.roll(ry1_0, -1, 2)
                    + _BF(8) * jnp.roll(ry1_0, -2, 2)
                    + _BF(4) * ry2_0 + _BF(2) * jnp.roll(ry2_0, -1, 2)
                    + jnp.roll(ry2_0, -2, 2))
            v_xy = v_xy * mxy_f + sxy_f
            axy = onehot_accum(axy, a_xy, v_xy, sign)
            rx1_1 = jnp.roll(Z1, -1, 2)
            rx2_1 = jnp.roll(Z1, -2, 2)
            rx1_2 = jnp.roll(Z2, -1, 2)
            rx2_2 = jnp.roll(Z2, -2, 2)
            a_yz = _BF(8) * Z0 + _BF(4) * rx1_0 + _BF(2) * rx2_0 + Z1
            v_yz = (_BF(16) * rx1_1 + _BF(8) * rx2_1 + _BF(4) * Z2
                    + _BF(2) * rx1_2 + rx2_2)
            v_yz = (v_yz * mx_f + sx_f) * mz + sz
            ayz = onehot_accum(ayz, a_yz, v_yz, sign)
            ry1_1 = jnp.roll(Z1, -1, 1)
            ry2_1 = jnp.roll(Z1, -2, 1)
            ry1_2 = jnp.roll(Z2, -1, 1)
            ry2_2 = jnp.roll(Z2, -2, 1)
            a_zx = _BF(8) * Z0 + _BF(4) * ry1_0 + _BF(2) * ry2_0 + Z1
            v_zx = (_BF(16) * ry1_1 + _BF(8) * ry2_1 + _BF(4) * Z2
                    + _BF(2) * ry1_2 + ry2_2)
            v_zx = (v_zx * my_f + sy_f) * mz + sz
            azx = onehot_accum(azx, a_zx, v_zx, sign)
        return (axy, ayz, azx)

    z0 = jnp.zeros((32, 16), jnp.float32)
    axy, ayz, azx = lax.fori_loop(0, 128, step, (z0, z0, z0))
    total = (jnp.sum(axy * axy) + jnp.sum(ayz * ayz) + jnp.sum(azx * azx))
    out_ref[:, :] = (total * _SCALE).reshape(1, 1)


@jax.jit
def kernel(input, target):
    out = pl.pallas_call(
        _pattern_kernel,
        out_shape=jax.ShapeDtypeStruct((1, 1), jnp.float32),
    )(input, target)
    return out[0, 0]
